# Initial kernel scaffold; baseline (speedup 1.0000x reference)
import jax
import jax.numpy as jnp
from jax.experimental import pallas as pl


def kernel(x, edge_index, batch, metadata, W1, b1, W2, b2, W3, b3, Wm, bm, Wp1, bp1, Wp2, bp2):
    raise NotImplementedError("write your pallas kernel here")


# speedup vs baseline: 18.5374x; 18.5374x over previous
import functools

import jax
import jax.numpy as jnp
from jax import lax
from jax.experimental import pallas as pl
from jax.experimental.pallas import tpu as pltpu
from jax.experimental.pallas import tpu_sc as plsc

N_NODES = 10000
N_EDGES = 320000
D_IN = 128
HID = 64
N_GRAPHS = 64

NC = 2
NS = 16
NW = NC * NS
CHUNK = 80
CPT = N_EDGES // NW // CHUNK
ROWS_PT = 624
ROWS_TAIL = N_NODES - NS * ROWS_PT
DEGW = 16

_mesh = plsc.VectorSubcoreMesh(core_axis_name="c", subcore_axis_name="s")


def _deg_body(dst_hbm, out_hbm, dst_v, ones_v, bounce_v, acc_sh):
    cid = lax.axis_index("c")
    sid = lax.axis_index("s")
    wid = cid * NS + sid
    pltpu.sync_copy(dst_hbm.at[wid], dst_v)

    def fill_ones(r, _):
        ones_v[r, pl.ds(0, DEGW)] = jnp.ones((16,), jnp.float32)
        return 0

    lax.fori_loop(0, CHUNK, fill_ones, 0)

    def fill_zero(r, _):
        bounce_v[r, pl.ds(0, DEGW)] = jnp.zeros((16,), jnp.float32)
        return 0

    lax.fori_loop(0, ROWS_PT, fill_zero, 0)
    pltpu.sync_copy(bounce_v, acc_sh.at[pl.ds(sid * ROWS_PT, ROWS_PT)])

    @pl.when(sid == 0)
    def _():
        pltpu.sync_copy(bounce_v.at[pl.ds(0, ROWS_TAIL)],
                        acc_sh.at[pl.ds(NS * ROWS_PT, ROWS_TAIL)])

    plsc.subcore_barrier()

    def step(j, _):
        pltpu.sync_copy(ones_v, acc_sh.at[dst_v.at[j]], add=True)
        return 0

    lax.fori_loop(0, CPT, step, 0)
    plsc.subcore_barrier()
    pltpu.sync_copy(acc_sh.at[pl.ds(sid * ROWS_PT, ROWS_PT)], bounce_v)
    pltpu.sync_copy(bounce_v, out_hbm.at[cid, pl.ds(sid * ROWS_PT, ROWS_PT)])

    @pl.when(sid == 0)
    def _():
        pltpu.sync_copy(acc_sh.at[pl.ds(NS * ROWS_PT, ROWS_TAIL)],
                        ones_v.at[pl.ds(0, ROWS_TAIL)])
        pltpu.sync_copy(ones_v.at[pl.ds(0, ROWS_TAIL)],
                        out_hbm.at[cid, pl.ds(NS * ROWS_PT, ROWS_TAIL)])


_deg_call = pl.kernel(
    _deg_body,
    out_type=jax.ShapeDtypeStruct((NC, N_NODES, DEGW), jnp.float32),
    mesh=_mesh,
    compiler_params=pltpu.CompilerParams(use_tc_tiling_on_sc=False),
    scratch_types=[
        pltpu.VMEM((CPT, CHUNK), jnp.int32),
        pltpu.VMEM((CHUNK, DEGW), jnp.float32),
        pltpu.VMEM((ROWS_PT, DEGW), jnp.float32),
        pltpu.VMEM_SHARED((N_NODES, DEGW), jnp.float32),
    ],
)


def _agg_body(g_hbm, src_hbm, dst_hbm, out_hbm,
              src_v, dst_v, rows_v, bounce_v, acc_sh, sem):
    cid = lax.axis_index("c")
    sid = lax.axis_index("s")
    wid = cid * NS + sid
    pltpu.sync_copy(src_hbm.at[wid], src_v)
    pltpu.sync_copy(dst_hbm.at[wid], dst_v)

    def fill_zero(r, _):
        for jj in range(HID // 16):
            bounce_v[r, pl.ds(jj * 16, 16)] = jnp.zeros((16,), jnp.float32)
        return 0

    lax.fori_loop(0, ROWS_PT, fill_zero, 0)
    pltpu.sync_copy(bounce_v, acc_sh.at[pl.ds(sid * ROWS_PT, ROWS_PT)])

    @pl.when(sid == 0)
    def _():
        pltpu.sync_copy(bounce_v.at[pl.ds(0, ROWS_TAIL)],
                        acc_sh.at[pl.ds(NS * ROWS_PT, ROWS_TAIL)])

    plsc.subcore_barrier()

    def step(j, _):
        pltpu.async_copy(g_hbm.at[src_v.at[j]], rows_v, sem).wait()
        pltpu.sync_copy(rows_v, acc_sh.at[dst_v.at[j]], add=True)
        return 0

    lax.fori_loop(0, CPT, step, 0)
    plsc.subcore_barrier()
    pltpu.sync_copy(acc_sh.at[pl.ds(sid * ROWS_PT, ROWS_PT)], bounce_v)
    pltpu.sync_copy(bounce_v, out_hbm.at[cid, pl.ds(sid * ROWS_PT, ROWS_PT)])

    @pl.when(sid == 0)
    def _():
        pltpu.sync_copy(acc_sh.at[pl.ds(NS * ROWS_PT, ROWS_TAIL)],
                        rows_v.at[pl.ds(0, ROWS_TAIL)])
        pltpu.sync_copy(rows_v.at[pl.ds(0, ROWS_TAIL)],
                        out_hbm.at[cid, pl.ds(NS * ROWS_PT, ROWS_TAIL)])


_agg_call = pl.kernel(
    _agg_body,
    out_type=jax.ShapeDtypeStruct((NC, N_NODES, HID), jnp.float32),
    mesh=_mesh,
    compiler_params=pltpu.CompilerParams(use_tc_tiling_on_sc=False),
    scratch_types=[
        pltpu.VMEM((CPT, CHUNK), jnp.int32),
        pltpu.VMEM((CPT, CHUNK), jnp.int32),
        pltpu.VMEM((CHUNK, HID), jnp.float32),
        pltpu.VMEM((ROWS_PT, HID), jnp.float32),
        pltpu.VMEM_SHARED((N_NODES, HID), jnp.float32),
        pltpu.SemaphoreType.DMA,
    ],
)


RBLK = 2000


def _first_body(x_ref, w_ref, deg_ref, g_ref, dinv_ref):
    deg = deg_ref[0, :, 0:1] + deg_ref[1, :, 0:1] + 1.0
    dinv = lax.rsqrt(deg)
    g_ref[...] = jnp.dot(x_ref[...], w_ref[...],
                         preferred_element_type=jnp.float32) * dinv
    dinv_ref[...] = dinv


def _tc_first(x, W1, degp):
    return pl.pallas_call(
        _first_body,
        grid=(N_NODES // RBLK,),
        in_specs=[
            pl.BlockSpec((RBLK, D_IN), lambda i: (i, 0)),
            pl.BlockSpec((D_IN, HID), lambda i: (0, 0)),
            pl.BlockSpec((NC, RBLK, DEGW), lambda i: (0, i, 0)),
        ],
        out_specs=[
            pl.BlockSpec((RBLK, HID), lambda i: (i, 0)),
            pl.BlockSpec((RBLK, 1), lambda i: (i, 0)),
        ],
        out_shape=[
            jax.ShapeDtypeStruct((N_NODES, HID), jnp.float32),
            jax.ShapeDtypeStruct((N_NODES, 1), jnp.float32),
        ],
    )(x, W1, degp)


def _mid_body(acc_ref, g_ref, dinv_ref, b_ref, w_ref, out_ref):
    s = acc_ref[0] + acc_ref[1] + g_ref[...]
    z = jnp.maximum(s * dinv_ref[...] + b_ref[...], 0.0)
    out_ref[...] = jnp.dot(z, w_ref[...],
                           preferred_element_type=jnp.float32) * dinv_ref[...]


def _tc_mid(acc, g_prev, dinv, b_row, W_next):
    return pl.pallas_call(
        _mid_body,
        grid=(N_NODES // RBLK,),
        in_specs=[
            pl.BlockSpec((NC, RBLK, HID), lambda i: (0, i, 0)),
            pl.BlockSpec((RBLK, HID), lambda i: (i, 0)),
            pl.BlockSpec((RBLK, 1), lambda i: (i, 0)),
            pl.BlockSpec((1, HID), lambda i: (0, 0)),
            pl.BlockSpec((HID, HID), lambda i: (0, 0)),
        ],
        out_specs=pl.BlockSpec((RBLK, HID), lambda i: (i, 0)),
        out_shape=jax.ShapeDtypeStruct((N_NODES, HID), jnp.float32),
    )(acc, g_prev, dinv, b_row, W_next)


NGRID = N_NODES // RBLK


def _head_body(acc_ref, g_ref, dinv_ref, b_ref, batch_ref, meta_ref,
               wm_ref, bm_ref, wp1_ref, bp1_ref, wp2_ref, bp2_ref, out_ref,
               sums_s, counts_s, maxes_s):
    i = pl.program_id(0)
    s = acc_ref[0] + acc_ref[1] + g_ref[...]
    z = jnp.maximum(s * dinv_ref[...] + b_ref[...], 0.0)
    gids = lax.broadcasted_iota(jnp.int32, (1, N_GRAPHS), 1)
    onehot = (batch_ref[...] == gids).astype(jnp.float32)
    bsums = lax.dot_general(onehot, z, (((0,), (0,)), ((), ())),
                            preferred_element_type=jnp.float32)
    bcounts = lax.dot_general(onehot, jnp.ones((RBLK, 1), jnp.float32),
                              (((0,), (0,)), ((), ())),
                              preferred_element_type=jnp.float32)
    rows = []
    neg = jnp.float32(-3.0e38)
    for g in range(N_GRAPHS):
        mask = batch_ref[...] == g
        m = jnp.max(jnp.where(mask, z, neg), axis=0, keepdims=True)
        rows.append(m)
    bmax = jnp.concatenate(rows, axis=0)

    @pl.when(i == 0)
    def _():
        sums_s[...] = bsums
        counts_s[...] = bcounts
        maxes_s[...] = bmax

    @pl.when(i > 0)
    def _():
        sums_s[...] += bsums
        counts_s[...] += bcounts
        maxes_s[...] = jnp.maximum(maxes_s[...], bmax)

    @pl.when(i == NGRID - 1)
    def _():
        counts = counts_s[...]
        mean = sums_s[...] / jnp.maximum(counts, 1.0)
        mx = jnp.where(counts > 0, maxes_s[...], 0.0)
        meta_e = jnp.maximum(
            jnp.dot(meta_ref[...], wm_ref[...],
                    preferred_element_type=jnp.float32) + bm_ref[...], 0.0)
        fused = jnp.concatenate([mean, mx, meta_e], axis=1)
        hp = jnp.maximum(
            jnp.dot(fused, wp1_ref[...],
                    preferred_element_type=jnp.float32) + bp1_ref[...], 0.0)
        out_ref[...] = jnp.dot(hp, wp2_ref[...],
                               preferred_element_type=jnp.float32) + bp2_ref[...]


def _tc_head(acc, g3, dinv, b_row, batch2d, metadata, Wm, bm_row,
             Wp1, bp1_row, Wp2, bp2_row):
    return pl.pallas_call(
        _head_body,
        grid=(NGRID,),
        in_specs=[
            pl.BlockSpec((NC, RBLK, HID), lambda i: (0, i, 0)),
            pl.BlockSpec((RBLK, HID), lambda i: (i, 0)),
            pl.BlockSpec((RBLK, 1), lambda i: (i, 0)),
            pl.BlockSpec((1, HID), lambda i: (0, 0)),
            pl.BlockSpec((RBLK, 1), lambda i: (i, 0)),
            pl.BlockSpec((N_GRAPHS, 32), lambda i: (0, 0)),
            pl.BlockSpec((32, HID), lambda i: (0, 0)),
            pl.BlockSpec((1, HID), lambda i: (0, 0)),
            pl.BlockSpec((2 * HID + 64, 64), lambda i: (0, 0)),
            pl.BlockSpec((1, 64), lambda i: (0, 0)),
            pl.BlockSpec((64, 1), lambda i: (0, 0)),
            pl.BlockSpec((1, 1), lambda i: (0, 0)),
        ],
        out_specs=pl.BlockSpec((N_GRAPHS, 1), lambda i: (0, 0)),
        out_shape=jax.ShapeDtypeStruct((N_GRAPHS, 1), jnp.float32),
        scratch_shapes=[
            pltpu.VMEM((N_GRAPHS, HID), jnp.float32),
            pltpu.VMEM((N_GRAPHS, 1), jnp.float32),
            pltpu.VMEM((N_GRAPHS, HID), jnp.float32),
        ],
    )(acc, g3, dinv, b_row, batch2d, metadata, Wm, bm_row,
      Wp1, bp1_row, Wp2, bp2_row)


def kernel(x, edge_index, batch, metadata, W1, b1, W2, b2, W3, b3,
           Wm, bm, Wp1, bp1, Wp2, bp2):
    src_r = edge_index[0].reshape(NW, CPT, CHUNK)
    dst_r = edge_index[1].reshape(NW, CPT, CHUNK)
    batch2d = batch.reshape(N_NODES, 1)
    b1r = b1.reshape(1, HID)
    b2r = b2.reshape(1, HID)
    b3r = b3.reshape(1, HID)
    bmr = bm.reshape(1, HID)
    bp1r = bp1.reshape(1, 64)
    bp2r = bp2.reshape(1, 1)

    degp = _deg_call(dst_r)
    g1, dinv = _tc_first(x, W1, degp)
    acc1 = _agg_call(g1, src_r, dst_r)
    g2 = _tc_mid(acc1, g1, dinv, b1r, W2)
    acc2 = _agg_call(g2, src_r, dst_r)
    g3 = _tc_mid(acc2, g2, dinv, b2r, W3)
    acc3 = _agg_call(g3, src_r, dst_r)
    return _tc_head(acc3, g3, dinv, b3r, batch2d, metadata, Wm, bmr,
                    Wp1, bp1r, Wp2, bp2r)


# speedup vs baseline: 29.4156x; 1.5868x over previous
import functools

import jax
import jax.numpy as jnp
from jax import lax
from jax.experimental import pallas as pl
from jax.experimental.pallas import tpu as pltpu
from jax.experimental.pallas import tpu_sc as plsc

N_NODES = 10000
N_EDGES = 320000
D_IN = 128
HID = 64
N_GRAPHS = 64

NC = 2
NS = 16
NW = NC * NS
CHUNK = 80
CPT = N_EDGES // NW // CHUNK
ROWS_PT = 624
ROWS_TAIL = N_NODES - NS * ROWS_PT
DEGW = 16
NBUF = 4

_mesh = plsc.VectorSubcoreMesh(core_axis_name="c", subcore_axis_name="s")


def _deg_body(dst_hbm, out_hbm, dst_v, ones_v, bounce_v, acc_sh):
    cid = lax.axis_index("c")
    sid = lax.axis_index("s")
    wid = cid * NS + sid
    pltpu.sync_copy(dst_hbm.at[wid], dst_v)

    def fill_ones(r, _):
        ones_v[r, pl.ds(0, DEGW)] = jnp.ones((16,), jnp.float32)
        return 0

    lax.fori_loop(0, CHUNK, fill_ones, 0)

    def fill_zero(r, _):
        bounce_v[r, pl.ds(0, DEGW)] = jnp.zeros((16,), jnp.float32)
        return 0

    lax.fori_loop(0, ROWS_PT, fill_zero, 0)
    pltpu.sync_copy(bounce_v, acc_sh.at[pl.ds(sid * ROWS_PT, ROWS_PT)])

    @pl.when(sid == 0)
    def _():
        pltpu.sync_copy(bounce_v.at[pl.ds(0, ROWS_TAIL)],
                        acc_sh.at[pl.ds(NS * ROWS_PT, ROWS_TAIL)])

    plsc.subcore_barrier()

    def step(j, _):
        pltpu.sync_copy(ones_v, acc_sh.at[dst_v.at[j]], add=True)
        return 0

    lax.fori_loop(0, CPT, step, 0)
    plsc.subcore_barrier()
    pltpu.sync_copy(acc_sh.at[pl.ds(sid * ROWS_PT, ROWS_PT)], bounce_v)
    pltpu.sync_copy(bounce_v, out_hbm.at[cid, pl.ds(sid * ROWS_PT, ROWS_PT)])

    @pl.when(sid == 0)
    def _():
        pltpu.sync_copy(acc_sh.at[pl.ds(NS * ROWS_PT, ROWS_TAIL)],
                        ones_v.at[pl.ds(0, ROWS_TAIL)])
        pltpu.sync_copy(ones_v.at[pl.ds(0, ROWS_TAIL)],
                        out_hbm.at[cid, pl.ds(NS * ROWS_PT, ROWS_TAIL)])


_deg_call = pl.kernel(
    _deg_body,
    out_type=jax.ShapeDtypeStruct((NC, N_NODES, DEGW), jnp.float32),
    mesh=_mesh,
    compiler_params=pltpu.CompilerParams(use_tc_tiling_on_sc=False),
    scratch_types=[
        pltpu.VMEM((CPT, CHUNK), jnp.int32),
        pltpu.VMEM((CHUNK, DEGW), jnp.float32),
        pltpu.VMEM((ROWS_PT, DEGW), jnp.float32),
        pltpu.VMEM_SHARED((N_NODES, DEGW), jnp.float32),
    ],
)


def _agg_body(g_hbm, src_hbm, dst_hbm, out_hbm,
              src_v, dst_v, rows_v, bounce_v, acc_sh, gsem, ssem):
    cid = lax.axis_index("c")
    sid = lax.axis_index("s")
    wid = cid * NS + sid
    pltpu.sync_copy(src_hbm.at[wid], src_v)
    pltpu.sync_copy(dst_hbm.at[wid], dst_v)

    def fill_zero(r, _):
        for jj in range(HID // 16):
            bounce_v[r, pl.ds(jj * 16, 16)] = jnp.zeros((16,), jnp.float32)
        return 0

    lax.fori_loop(0, ROWS_PT, fill_zero, 0)
    pltpu.sync_copy(bounce_v, acc_sh.at[pl.ds(sid * ROWS_PT, ROWS_PT)])

    @pl.when(sid == 0)
    def _():
        pltpu.sync_copy(bounce_v.at[pl.ds(0, ROWS_TAIL)],
                        acc_sh.at[pl.ds(NS * ROWS_PT, ROWS_TAIL)])

    plsc.subcore_barrier()

    def start_gather(j, p):
        pltpu.async_copy(g_hbm.at[src_v.at[j]], rows_v.at[p], gsem.at[p])

    def wait_dma(p, sem):
        pltpu.make_async_copy(g_hbm.at[src_v.at[0]], rows_v.at[p],
                              sem.at[p]).wait()

    def start_scatter(j, p):
        pltpu.async_copy(rows_v.at[p], acc_sh.at[dst_v.at[j]], ssem.at[p],
                         add=True)

    for p in range(NBUF):
        start_gather(p, p)

    def step(i, _):
        for p in range(NBUF):
            j = NBUF * i + p
            wait_dma(p, gsem)
            start_scatter(j, p)
        for p in range(NBUF):
            jn = NBUF * i + p + NBUF

            @pl.when(jn <= CPT - 1)
            def _():
                wait_dma(p, ssem)
                start_gather(jn, p)

        return 0

    lax.fori_loop(0, (CPT - 1) // NBUF, step, 0)
    wait_dma(1, ssem)
    wait_dma(2, ssem)
    wait_dma(3, ssem)
    wait_dma(0, gsem)
    start_scatter(CPT - 1, 0)
    wait_dma(0, ssem)
    plsc.subcore_barrier()
    pltpu.sync_copy(acc_sh.at[pl.ds(sid * ROWS_PT, ROWS_PT)], bounce_v)
    pltpu.sync_copy(bounce_v, out_hbm.at[cid, pl.ds(sid * ROWS_PT, ROWS_PT)])

    @pl.when(sid == 0)
    def _():
        pltpu.sync_copy(acc_sh.at[pl.ds(NS * ROWS_PT, ROWS_TAIL)],
                        bounce_v.at[pl.ds(0, ROWS_TAIL)])
        pltpu.sync_copy(bounce_v.at[pl.ds(0, ROWS_TAIL)],
                        out_hbm.at[cid, pl.ds(NS * ROWS_PT, ROWS_TAIL)])


_agg_call = pl.kernel(
    _agg_body,
    out_type=jax.ShapeDtypeStruct((NC, N_NODES, HID), jnp.float32),
    mesh=_mesh,
    compiler_params=pltpu.CompilerParams(use_tc_tiling_on_sc=False),
    scratch_types=[
        pltpu.VMEM((CPT, CHUNK), jnp.int32),
        pltpu.VMEM((CPT, CHUNK), jnp.int32),
        pltpu.VMEM((NBUF, CHUNK, HID), jnp.float32),
        pltpu.VMEM((ROWS_PT, HID), jnp.float32),
        pltpu.VMEM_SHARED((N_NODES, HID), jnp.float32),
        pltpu.SemaphoreType.DMA((NBUF,)),
        pltpu.SemaphoreType.DMA((NBUF,)),
    ],
)


RBLK = 2000


def _first_body(x_ref, w_ref, deg_ref, g_ref, dinv_ref):
    deg = deg_ref[0, :, 0:1] + deg_ref[1, :, 0:1] + 1.0
    dinv = lax.rsqrt(deg)
    g_ref[...] = jnp.dot(x_ref[...], w_ref[...],
                         preferred_element_type=jnp.float32) * dinv
    dinv_ref[...] = dinv


def _tc_first(x, W1, degp):
    return pl.pallas_call(
        _first_body,
        grid=(N_NODES // RBLK,),
        in_specs=[
            pl.BlockSpec((RBLK, D_IN), lambda i: (i, 0)),
            pl.BlockSpec((D_IN, HID), lambda i: (0, 0)),
            pl.BlockSpec((NC, RBLK, DEGW), lambda i: (0, i, 0)),
        ],
        out_specs=[
            pl.BlockSpec((RBLK, HID), lambda i: (i, 0)),
            pl.BlockSpec((RBLK, 1), lambda i: (i, 0)),
        ],
        out_shape=[
            jax.ShapeDtypeStruct((N_NODES, HID), jnp.float32),
            jax.ShapeDtypeStruct((N_NODES, 1), jnp.float32),
        ],
    )(x, W1, degp)


def _mid_body(acc_ref, g_ref, dinv_ref, b_ref, w_ref, out_ref):
    s = acc_ref[0] + acc_ref[1] + g_ref[...]
    z = jnp.maximum(s * dinv_ref[...] + b_ref[...], 0.0)
    out_ref[...] = jnp.dot(z, w_ref[...],
                           preferred_element_type=jnp.float32) * dinv_ref[...]


def _tc_mid(acc, g_prev, dinv, b_row, W_next):
    return pl.pallas_call(
        _mid_body,
        grid=(N_NODES // RBLK,),
        in_specs=[
            pl.BlockSpec((NC, RBLK, HID), lambda i: (0, i, 0)),
            pl.BlockSpec((RBLK, HID), lambda i: (i, 0)),
            pl.BlockSpec((RBLK, 1), lambda i: (i, 0)),
            pl.BlockSpec((1, HID), lambda i: (0, 0)),
            pl.BlockSpec((HID, HID), lambda i: (0, 0)),
        ],
        out_specs=pl.BlockSpec((RBLK, HID), lambda i: (i, 0)),
        out_shape=jax.ShapeDtypeStruct((N_NODES, HID), jnp.float32),
    )(acc, g_prev, dinv, b_row, W_next)


NGRID = N_NODES // RBLK


def _head_body(acc_ref, g_ref, dinv_ref, b_ref, batch_ref, meta_ref,
               wm_ref, bm_ref, wp1_ref, bp1_ref, wp2_ref, bp2_ref, out_ref,
               sums_s, counts_s, maxes_s):
    i = pl.program_id(0)
    s = acc_ref[0] + acc_ref[1] + g_ref[...]
    z = jnp.maximum(s * dinv_ref[...] + b_ref[...], 0.0)
    gids = lax.broadcasted_iota(jnp.int32, (1, N_GRAPHS), 1)
    onehot = (batch_ref[...] == gids).astype(jnp.float32)
    bsums = lax.dot_general(onehot, z, (((0,), (0,)), ((), ())),
                            preferred_element_type=jnp.float32)
    bcounts = lax.dot_general(onehot, jnp.ones((RBLK, 1), jnp.float32),
                              (((0,), (0,)), ((), ())),
                              preferred_element_type=jnp.float32)
    rows = []
    neg = jnp.float32(-3.0e38)
    for g in range(N_GRAPHS):
        mask = batch_ref[...] == g
        m = jnp.max(jnp.where(mask, z, neg), axis=0, keepdims=True)
        rows.append(m)
    bmax = jnp.concatenate(rows, axis=0)

    @pl.when(i == 0)
    def _():
        sums_s[...] = bsums
        counts_s[...] = bcounts
        maxes_s[...] = bmax

    @pl.when(i > 0)
    def _():
        sums_s[...] += bsums
        counts_s[...] += bcounts
        maxes_s[...] = jnp.maximum(maxes_s[...], bmax)

    @pl.when(i == NGRID - 1)
    def _():
        counts = counts_s[...]
        mean = sums_s[...] / jnp.maximum(counts, 1.0)
        mx = jnp.where(counts > 0, maxes_s[...], 0.0)
        meta_e = jnp.maximum(
            jnp.dot(meta_ref[...], wm_ref[...],
                    preferred_element_type=jnp.float32) + bm_ref[...], 0.0)
        fused = jnp.concatenate([mean, mx, meta_e], axis=1)
        hp = jnp.maximum(
            jnp.dot(fused, wp1_ref[...],
                    preferred_element_type=jnp.float32) + bp1_ref[...], 0.0)
        out_ref[...] = jnp.dot(hp, wp2_ref[...],
                               preferred_element_type=jnp.float32) + bp2_ref[...]


def _tc_head(acc, g3, dinv, b_row, batch2d, metadata, Wm, bm_row,
             Wp1, bp1_row, Wp2, bp2_row):
    return pl.pallas_call(
        _head_body,
        grid=(NGRID,),
        in_specs=[
            pl.BlockSpec((NC, RBLK, HID), lambda i: (0, i, 0)),
            pl.BlockSpec((RBLK, HID), lambda i: (i, 0)),
            pl.BlockSpec((RBLK, 1), lambda i: (i, 0)),
            pl.BlockSpec((1, HID), lambda i: (0, 0)),
            pl.BlockSpec((RBLK, 1), lambda i: (i, 0)),
            pl.BlockSpec((N_GRAPHS, 32), lambda i: (0, 0)),
            pl.BlockSpec((32, HID), lambda i: (0, 0)),
            pl.BlockSpec((1, HID), lambda i: (0, 0)),
            pl.BlockSpec((2 * HID + 64, 64), lambda i: (0, 0)),
            pl.BlockSpec((1, 64), lambda i: (0, 0)),
            pl.BlockSpec((64, 1), lambda i: (0, 0)),
            pl.BlockSpec((1, 1), lambda i: (0, 0)),
        ],
        out_specs=pl.BlockSpec((N_GRAPHS, 1), lambda i: (0, 0)),
        out_shape=jax.ShapeDtypeStruct((N_GRAPHS, 1), jnp.float32),
        scratch_shapes=[
            pltpu.VMEM((N_GRAPHS, HID), jnp.float32),
            pltpu.VMEM((N_GRAPHS, 1), jnp.float32),
            pltpu.VMEM((N_GRAPHS, HID), jnp.float32),
        ],
    )(acc, g3, dinv, b_row, batch2d, metadata, Wm, bm_row,
      Wp1, bp1_row, Wp2, bp2_row)


def kernel(x, edge_index, batch, metadata, W1, b1, W2, b2, W3, b3,
           Wm, bm, Wp1, bp1, Wp2, bp2):
    src_r = edge_index[0].reshape(NW, CPT, CHUNK)
    dst_r = edge_index[1].reshape(NW, CPT, CHUNK)
    batch2d = batch.reshape(N_NODES, 1)
    b1r = b1.reshape(1, HID)
    b2r = b2.reshape(1, HID)
    b3r = b3.reshape(1, HID)
    bmr = bm.reshape(1, HID)
    bp1r = bp1.reshape(1, 64)
    bp2r = bp2.reshape(1, 1)

    degp = _deg_call(dst_r)
    g1, dinv = _tc_first(x, W1, degp)
    acc1 = _agg_call(g1, src_r, dst_r)
    g2 = _tc_mid(acc1, g1, dinv, b1r, W2)
    acc2 = _agg_call(g2, src_r, dst_r)
    g3 = _tc_mid(acc2, g2, dinv, b2r, W3)
    acc3 = _agg_call(g3, src_r, dst_r)
    return _tc_head(acc3, g3, dinv, b3r, batch2d, metadata, Wm, bmr,
                    Wp1, bp1r, Wp2, bp2r)


# speedup vs baseline: 35.4752x; 1.2060x over previous
import functools

import jax
import jax.numpy as jnp
from jax import lax
from jax.experimental import pallas as pl
from jax.experimental.pallas import tpu as pltpu
from jax.experimental.pallas import tpu_sc as plsc

N_NODES = 10000
N_EDGES = 320000
D_IN = 128
HID = 64
N_GRAPHS = 64

NC = 2
NS = 16
NW = NC * NS
CHUNK = 80
CPT = N_EDGES // NW // CHUNK
ROWS_PT = 624
ROWS_TAIL = N_NODES - NS * ROWS_PT
DEGW = 16
NBUF = 4

_mesh = plsc.VectorSubcoreMesh(core_axis_name="c", subcore_axis_name="s")


def _deg_body(dst_hbm, out_hbm, dst_v, ones_v, bounce_v, acc_sh):
    cid = lax.axis_index("c")
    sid = lax.axis_index("s")
    wid = cid * NS + sid
    pltpu.sync_copy(dst_hbm.at[wid], dst_v)

    def fill_ones(r, _):
        ones_v[r, pl.ds(0, DEGW)] = jnp.ones((16,), jnp.float32)
        return 0

    lax.fori_loop(0, CHUNK, fill_ones, 0)

    def fill_zero(r, _):
        bounce_v[r, pl.ds(0, DEGW)] = jnp.zeros((16,), jnp.float32)
        return 0

    lax.fori_loop(0, ROWS_PT, fill_zero, 0)
    pltpu.sync_copy(bounce_v, acc_sh.at[pl.ds(sid * ROWS_PT, ROWS_PT)])

    @pl.when(sid == 0)
    def _():
        pltpu.sync_copy(bounce_v.at[pl.ds(0, ROWS_TAIL)],
                        acc_sh.at[pl.ds(NS * ROWS_PT, ROWS_TAIL)])

    plsc.subcore_barrier()

    def step(j, _):
        pltpu.sync_copy(ones_v, acc_sh.at[dst_v.at[j]], add=True)
        return 0

    lax.fori_loop(0, CPT, step, 0)
    plsc.subcore_barrier()
    pltpu.sync_copy(acc_sh.at[pl.ds(sid * ROWS_PT, ROWS_PT)], bounce_v)
    pltpu.sync_copy(bounce_v, out_hbm.at[cid, pl.ds(sid * ROWS_PT, ROWS_PT)])

    @pl.when(sid == 0)
    def _():
        pltpu.sync_copy(acc_sh.at[pl.ds(NS * ROWS_PT, ROWS_TAIL)],
                        ones_v.at[pl.ds(0, ROWS_TAIL)])
        pltpu.sync_copy(ones_v.at[pl.ds(0, ROWS_TAIL)],
                        out_hbm.at[cid, pl.ds(NS * ROWS_PT, ROWS_TAIL)])


_deg_call = pl.kernel(
    _deg_body,
    out_type=jax.ShapeDtypeStruct((NC, N_NODES, DEGW), jnp.float32),
    mesh=_mesh,
    compiler_params=pltpu.CompilerParams(use_tc_tiling_on_sc=False),
    scratch_types=[
        pltpu.VMEM((CPT, CHUNK), jnp.int32),
        pltpu.VMEM((CHUNK, DEGW), jnp.float32),
        pltpu.VMEM((ROWS_PT, DEGW), jnp.float32),
        pltpu.VMEM_SHARED((N_NODES, DEGW), jnp.float32),
    ],
)


def _agg_body(g_hbm, src_hbm, dst_hbm, out_hbm,
              src_v, dst_v, rows_v, bounce_v, acc_sh, gsem, ssem):
    cid = lax.axis_index("c")
    sid = lax.axis_index("s")
    wid = cid * NS + sid
    pltpu.sync_copy(src_hbm.at[wid], src_v)
    pltpu.sync_copy(dst_hbm.at[wid], dst_v)

    def fill_zero(r, _):
        for jj in range(HID // 16):
            bounce_v[r, pl.ds(jj * 16, 16)] = jnp.zeros((16,), jnp.float32)
        return 0

    lax.fori_loop(0, ROWS_PT, fill_zero, 0)
    pltpu.sync_copy(bounce_v, acc_sh.at[pl.ds(sid * ROWS_PT, ROWS_PT)])

    @pl.when(sid == 0)
    def _():
        pltpu.sync_copy(bounce_v.at[pl.ds(0, ROWS_TAIL)],
                        acc_sh.at[pl.ds(NS * ROWS_PT, ROWS_TAIL)])

    plsc.subcore_barrier()

    def start_gather(j, p):
        pltpu.async_copy(g_hbm.at[src_v.at[j]], rows_v.at[p], gsem.at[p])

    def wait_dma(p, sem):
        pltpu.make_async_copy(g_hbm.at[src_v.at[0]], rows_v.at[p],
                              sem.at[p]).wait()

    def start_scatter(j, p):
        pltpu.async_copy(rows_v.at[p], acc_sh.at[dst_v.at[j]], ssem.at[p],
                         add=True)

    for p in range(NBUF):
        start_gather(p, p)

    def step(i, _):
        for p in range(NBUF):
            j = NBUF * i + p
            wait_dma(p, gsem)
            start_scatter(j, p)
        for p in range(NBUF):
            jn = NBUF * i + p + NBUF

            @pl.when(jn <= CPT - 1)
            def _():
                wait_dma(p, ssem)
                start_gather(jn, p)

        return 0

    lax.fori_loop(0, (CPT - 1) // NBUF, step, 0)
    wait_dma(1, ssem)
    wait_dma(2, ssem)
    wait_dma(3, ssem)
    wait_dma(0, gsem)
    start_scatter(CPT - 1, 0)
    wait_dma(0, ssem)
    plsc.subcore_barrier()
    pltpu.sync_copy(acc_sh.at[pl.ds(sid * ROWS_PT, ROWS_PT)], bounce_v)
    pltpu.sync_copy(bounce_v, out_hbm.at[cid, pl.ds(sid * ROWS_PT, ROWS_PT)])

    @pl.when(sid == 0)
    def _():
        pltpu.sync_copy(acc_sh.at[pl.ds(NS * ROWS_PT, ROWS_TAIL)],
                        bounce_v.at[pl.ds(0, ROWS_TAIL)])
        pltpu.sync_copy(bounce_v.at[pl.ds(0, ROWS_TAIL)],
                        out_hbm.at[cid, pl.ds(NS * ROWS_PT, ROWS_TAIL)])


_agg_call = pl.kernel(
    _agg_body,
    out_type=jax.ShapeDtypeStruct((NC, N_NODES, HID), jnp.float32),
    mesh=_mesh,
    compiler_params=pltpu.CompilerParams(use_tc_tiling_on_sc=False),
    scratch_types=[
        pltpu.VMEM((CPT, CHUNK), jnp.int32),
        pltpu.VMEM((CPT, CHUNK), jnp.int32),
        pltpu.VMEM((NBUF, CHUNK, HID), jnp.float32),
        pltpu.VMEM((ROWS_PT, HID), jnp.float32),
        pltpu.VMEM_SHARED((N_NODES, HID), jnp.float32),
        pltpu.SemaphoreType.DMA((NBUF,)),
        pltpu.SemaphoreType.DMA((NBUF,)),
    ],
)


RBLK = 2000


def _first_body(x_ref, w_ref, deg_ref, g_ref, dinv_ref):
    deg = deg_ref[0, :, 0:1] + deg_ref[1, :, 0:1] + 1.0
    dinv = lax.rsqrt(deg)
    g_ref[...] = jnp.dot(x_ref[...], w_ref[...],
                         preferred_element_type=jnp.float32) * dinv
    dinv_ref[...] = dinv


def _tc_first(x, W1, degp):
    return pl.pallas_call(
        _first_body,
        grid=(N_NODES // RBLK,),
        in_specs=[
            pl.BlockSpec((RBLK, D_IN), lambda i: (i, 0)),
            pl.BlockSpec((D_IN, HID), lambda i: (0, 0)),
            pl.BlockSpec((NC, RBLK, DEGW), lambda i: (0, i, 0)),
        ],
        out_specs=[
            pl.BlockSpec((RBLK, HID), lambda i: (i, 0)),
            pl.BlockSpec((RBLK, 1), lambda i: (i, 0)),
        ],
        out_shape=[
            jax.ShapeDtypeStruct((N_NODES, HID), jnp.float32),
            jax.ShapeDtypeStruct((N_NODES, 1), jnp.float32),
        ],
    )(x, W1, degp)


def _mid_body(acc_ref, g_ref, dinv_ref, b_ref, w_ref, out_ref):
    s = acc_ref[0] + acc_ref[1] + g_ref[...]
    z = jnp.maximum(s * dinv_ref[...] + b_ref[...], 0.0)
    out_ref[...] = jnp.dot(z, w_ref[...],
                           preferred_element_type=jnp.float32) * dinv_ref[...]


def _tc_mid(acc, g_prev, dinv, b_row, W_next):
    return pl.pallas_call(
        _mid_body,
        grid=(N_NODES // RBLK,),
        in_specs=[
            pl.BlockSpec((NC, RBLK, HID), lambda i: (0, i, 0)),
            pl.BlockSpec((RBLK, HID), lambda i: (i, 0)),
            pl.BlockSpec((RBLK, 1), lambda i: (i, 0)),
            pl.BlockSpec((1, HID), lambda i: (0, 0)),
            pl.BlockSpec((HID, HID), lambda i: (0, 0)),
        ],
        out_specs=pl.BlockSpec((RBLK, HID), lambda i: (i, 0)),
        out_shape=jax.ShapeDtypeStruct((N_NODES, HID), jnp.float32),
    )(acc, g_prev, dinv, b_row, W_next)


ROWS_W = 320
NGROUPS = ROWS_W // 16
TAIL_ROWS = N_NODES - (NW - 1) * ROWS_W
TAIL_OFF = ROWS_W - TAIL_ROWS


def _pool_body(acc_hbm, g_hbm, dinv_hbm, b3_hbm, batch_hbm,
               sums_out, maxs_out, cnts_out,
               a0_v, a1_v, g_v, dinv_v, batch_v, b3_v, sum_t, max_t, cnt_t):
    cid = lax.axis_index("c")
    sid = lax.axis_index("s")
    w = cid * NS + sid
    is_last = w == NW - 1
    base = jnp.where(is_last, N_NODES - ROWS_W, w * ROWS_W)
    local_off = jnp.where(is_last, TAIL_OFF, 0)
    ngroups = jnp.where(is_last, TAIL_ROWS // 16, NGROUPS)
    pltpu.sync_copy(acc_hbm.at[0, pl.ds(base, ROWS_W)], a0_v)
    pltpu.sync_copy(acc_hbm.at[1, pl.ds(base, ROWS_W)], a1_v)
    pltpu.sync_copy(g_hbm.at[pl.ds(base, ROWS_W)], g_v)
    pltpu.sync_copy(dinv_hbm.at[pl.ds(base, ROWS_W)], dinv_v)
    pltpu.sync_copy(batch_hbm.at[pl.ds(base, ROWS_W)], batch_v)
    pltpu.sync_copy(b3_hbm, b3_v)

    def zt(r, _):
        for j in range(HID // 16):
            sum_t[r, pl.ds(16 * j, 16)] = jnp.zeros((16,), jnp.float32)
            max_t[r, pl.ds(16 * j, 16)] = jnp.zeros((16,), jnp.float32)
        cnt_t[r, pl.ds(0, 16)] = jnp.zeros((16,), jnp.float32)
        return 0

    lax.fori_loop(0, N_GRAPHS, zt, 0)
    ones16 = jnp.ones((16,), jnp.float32)

    def group(k, _):
        off = local_off + 16 * k
        bvec = batch_v[pl.ds(off, 16)]
        dvec = dinv_v[pl.ds(off, 16)]
        for t in range(16):
            b = bvec[t]
            dv = jnp.broadcast_to(dvec[t], (16,))
            r = off + t
            plsc.addupdate(cnt_t.at[b], ones16)
            for j in range(HID // 16):
                sl = pl.ds(16 * j, 16)
                s = a0_v[r, sl] + a1_v[r, sl] + g_v[r, sl]
                z = jnp.maximum(s * dv + b3_v[sl], 0.0)
                plsc.addupdate(sum_t.at[b, sl], z)
                max_t[b, sl] = jnp.maximum(max_t[b, sl], z)
        return 0

    lax.fori_loop(0, ngroups, group, 0)
    pltpu.sync_copy(sum_t, sums_out.at[w])
    pltpu.sync_copy(max_t, maxs_out.at[w])
    pltpu.sync_copy(cnt_t, cnts_out.at[w])


_pool_call = pl.kernel(
    _pool_body,
    out_type=[
        jax.ShapeDtypeStruct((NW, N_GRAPHS, HID), jnp.float32),
        jax.ShapeDtypeStruct((NW, N_GRAPHS, HID), jnp.float32),
        jax.ShapeDtypeStruct((NW, N_GRAPHS, 16), jnp.float32),
    ],
    mesh=_mesh,
    compiler_params=pltpu.CompilerParams(use_tc_tiling_on_sc=False),
    scratch_types=[
        pltpu.VMEM((ROWS_W, HID), jnp.float32),
        pltpu.VMEM((ROWS_W, HID), jnp.float32),
        pltpu.VMEM((ROWS_W, HID), jnp.float32),
        pltpu.VMEM((ROWS_W,), jnp.float32),
        pltpu.VMEM((ROWS_W,), jnp.int32),
        pltpu.VMEM((HID,), jnp.float32),
        pltpu.VMEM((N_GRAPHS, HID), jnp.float32),
        pltpu.VMEM((N_GRAPHS, HID), jnp.float32),
        pltpu.VMEM((N_GRAPHS, 16), jnp.float32),
    ],
)


def _head2_body(sums_ref, maxs_ref, cnts_ref, meta_ref, wm_ref, bm_ref,
                wp1_ref, bp1_ref, wp2_ref, bp2_ref, out_ref):
    sums = jnp.sum(sums_ref[...], axis=0)
    mx = jnp.max(maxs_ref[...], axis=0)
    counts = jnp.sum(cnts_ref[...], axis=0)[:, 0:1]
    mean = sums / jnp.maximum(counts, 1.0)
    meta_e = jnp.maximum(
        jnp.dot(meta_ref[...], wm_ref[...],
                preferred_element_type=jnp.float32) + bm_ref[...], 0.0)
    fused = jnp.concatenate([mean, mx, meta_e], axis=1)
    hp = jnp.maximum(
        jnp.dot(fused, wp1_ref[...],
                preferred_element_type=jnp.float32) + bp1_ref[...], 0.0)
    out_ref[...] = jnp.dot(hp, wp2_ref[...],
                           preferred_element_type=jnp.float32) + bp2_ref[...]


def _tc_head2(sums_p, maxs_p, cnts_p, metadata, Wm, bm_row,
              Wp1, bp1_row, Wp2, bp2_row):
    return pl.pallas_call(
        _head2_body,
        out_shape=jax.ShapeDtypeStruct((N_GRAPHS, 1), jnp.float32),
    )(sums_p, maxs_p, cnts_p, metadata, Wm, bm_row,
      Wp1, bp1_row, Wp2, bp2_row)


NGRID = N_NODES // RBLK


def _head_body(acc_ref, g_ref, dinv_ref, b_ref, batch_ref, meta_ref,
               wm_ref, bm_ref, wp1_ref, bp1_ref, wp2_ref, bp2_ref, out_ref,
               sums_s, counts_s, maxes_s):
    i = pl.program_id(0)
    s = acc_ref[0] + acc_ref[1] + g_ref[...]
    z = jnp.maximum(s * dinv_ref[...] + b_ref[...], 0.0)
    gids = lax.broadcasted_iota(jnp.int32, (1, N_GRAPHS), 1)
    onehot = (batch_ref[...] == gids).astype(jnp.float32)
    bsums = lax.dot_general(onehot, z, (((0,), (0,)), ((), ())),
                            preferred_element_type=jnp.float32)
    bcounts = lax.dot_general(onehot, jnp.ones((RBLK, 1), jnp.float32),
                              (((0,), (0,)), ((), ())),
                              preferred_element_type=jnp.float32)
    rows = []
    neg = jnp.float32(-3.0e38)
    for g in range(N_GRAPHS):
        mask = batch_ref[...] == g
        m = jnp.max(jnp.where(mask, z, neg), axis=0, keepdims=True)
        rows.append(m)
    bmax = jnp.concatenate(rows, axis=0)

    @pl.when(i == 0)
    def _():
        sums_s[...] = bsums
        counts_s[...] = bcounts
        maxes_s[...] = bmax

    @pl.when(i > 0)
    def _():
        sums_s[...] += bsums
        counts_s[...] += bcounts
        maxes_s[...] = jnp.maximum(maxes_s[...], bmax)

    @pl.when(i == NGRID - 1)
    def _():
        counts = counts_s[...]
        mean = sums_s[...] / jnp.maximum(counts, 1.0)
        mx = jnp.where(counts > 0, maxes_s[...], 0.0)
        meta_e = jnp.maximum(
            jnp.dot(meta_ref[...], wm_ref[...],
                    preferred_element_type=jnp.float32) + bm_ref[...], 0.0)
        fused = jnp.concatenate([mean, mx, meta_e], axis=1)
        hp = jnp.maximum(
            jnp.dot(fused, wp1_ref[...],
                    preferred_element_type=jnp.float32) + bp1_ref[...], 0.0)
        out_ref[...] = jnp.dot(hp, wp2_ref[...],
                               preferred_element_type=jnp.float32) + bp2_ref[...]


def _tc_head(acc, g3, dinv, b_row, batch2d, metadata, Wm, bm_row,
             Wp1, bp1_row, Wp2, bp2_row):
    return pl.pallas_call(
        _head_body,
        grid=(NGRID,),
        in_specs=[
            pl.BlockSpec((NC, RBLK, HID), lambda i: (0, i, 0)),
            pl.BlockSpec((RBLK, HID), lambda i: (i, 0)),
            pl.BlockSpec((RBLK, 1), lambda i: (i, 0)),
            pl.BlockSpec((1, HID), lambda i: (0, 0)),
            pl.BlockSpec((RBLK, 1), lambda i: (i, 0)),
            pl.BlockSpec((N_GRAPHS, 32), lambda i: (0, 0)),
            pl.BlockSpec((32, HID), lambda i: (0, 0)),
            pl.BlockSpec((1, HID), lambda i: (0, 0)),
            pl.BlockSpec((2 * HID + 64, 64), lambda i: (0, 0)),
            pl.BlockSpec((1, 64), lambda i: (0, 0)),
            pl.BlockSpec((64, 1), lambda i: (0, 0)),
            pl.BlockSpec((1, 1), lambda i: (0, 0)),
        ],
        out_specs=pl.BlockSpec((N_GRAPHS, 1), lambda i: (0, 0)),
        out_shape=jax.ShapeDtypeStruct((N_GRAPHS, 1), jnp.float32),
        scratch_shapes=[
            pltpu.VMEM((N_GRAPHS, HID), jnp.float32),
            pltpu.VMEM((N_GRAPHS, 1), jnp.float32),
            pltpu.VMEM((N_GRAPHS, HID), jnp.float32),
        ],
    )(acc, g3, dinv, b_row, batch2d, metadata, Wm, bm_row,
      Wp1, bp1_row, Wp2, bp2_row)


def kernel(x, edge_index, batch, metadata, W1, b1, W2, b2, W3, b3,
           Wm, bm, Wp1, bp1, Wp2, bp2):
    src_r = edge_index[0].reshape(NW, CPT, CHUNK)
    dst_r = edge_index[1].reshape(NW, CPT, CHUNK)
    batch2d = batch.reshape(N_NODES, 1)
    b1r = b1.reshape(1, HID)
    b2r = b2.reshape(1, HID)
    b3r = b3.reshape(1, HID)
    bmr = bm.reshape(1, HID)
    bp1r = bp1.reshape(1, 64)
    bp2r = bp2.reshape(1, 1)

    degp = _deg_call(dst_r)
    g1, dinv = _tc_first(x, W1, degp)
    acc1 = _agg_call(g1, src_r, dst_r)
    g2 = _tc_mid(acc1, g1, dinv, b1r, W2)
    acc2 = _agg_call(g2, src_r, dst_r)
    g3 = _tc_mid(acc2, g2, dinv, b2r, W3)
    acc3 = _agg_call(g3, src_r, dst_r)
    sums_p, maxs_p, cnts_p = _pool_call(acc3, g3, dinv.reshape(N_NODES),
                                        b3, batch)
    return _tc_head2(sums_p, maxs_p, cnts_p, metadata, Wm, bmr,
                     Wp1, bp1r, Wp2, bp2r)


# speedup vs baseline: 38.2427x; 1.0780x over previous
import functools

import jax
import jax.numpy as jnp
from jax import lax
from jax.experimental import pallas as pl
from jax.experimental.pallas import tpu as pltpu
from jax.experimental.pallas import tpu_sc as plsc

N_NODES = 10000
N_EDGES = 320000
D_IN = 128
HID = 64
N_GRAPHS = 64

NC = 2
NS = 16
NW = NC * NS
CHUNK = 80
CPT = N_EDGES // NW // CHUNK
ROWS_PT = 624
ROWS_TAIL = N_NODES - NS * ROWS_PT
DEGW = 16
NBUF = 8
NITER = (CPT - 1) // NBUF
TAILN = CPT - NBUF * NITER

_mesh = plsc.VectorSubcoreMesh(core_axis_name="c", subcore_axis_name="s")


def _deg_body(dst_hbm, out_hbm, dst_v, ones_v, bounce_v, acc_sh):
    cid = lax.axis_index("c")
    sid = lax.axis_index("s")
    wid = cid * NS + sid
    pltpu.sync_copy(dst_hbm.at[wid], dst_v)

    def fill_ones(r, _):
        ones_v[r, pl.ds(0, DEGW)] = jnp.ones((16,), jnp.float32)
        return 0

    lax.fori_loop(0, CHUNK, fill_ones, 0)

    def fill_zero(r, _):
        bounce_v[r, pl.ds(0, DEGW)] = jnp.zeros((16,), jnp.float32)
        return 0

    lax.fori_loop(0, ROWS_PT, fill_zero, 0)
    pltpu.sync_copy(bounce_v, acc_sh.at[pl.ds(sid * ROWS_PT, ROWS_PT)])

    @pl.when(sid == 0)
    def _():
        pltpu.sync_copy(bounce_v.at[pl.ds(0, ROWS_TAIL)],
                        acc_sh.at[pl.ds(NS * ROWS_PT, ROWS_TAIL)])

    plsc.subcore_barrier()

    def step(j, _):
        pltpu.sync_copy(ones_v, acc_sh.at[dst_v.at[j]], add=True)
        return 0

    lax.fori_loop(0, CPT, step, 0)
    plsc.subcore_barrier()
    pltpu.sync_copy(acc_sh.at[pl.ds(sid * ROWS_PT, ROWS_PT)], bounce_v)
    pltpu.sync_copy(bounce_v, out_hbm.at[cid, pl.ds(sid * ROWS_PT, ROWS_PT)])

    @pl.when(sid == 0)
    def _():
        pltpu.sync_copy(acc_sh.at[pl.ds(NS * ROWS_PT, ROWS_TAIL)],
                        ones_v.at[pl.ds(0, ROWS_TAIL)])
        pltpu.sync_copy(ones_v.at[pl.ds(0, ROWS_TAIL)],
                        out_hbm.at[cid, pl.ds(NS * ROWS_PT, ROWS_TAIL)])


_deg_call = pl.kernel(
    _deg_body,
    out_type=jax.ShapeDtypeStruct((NC, N_NODES, DEGW), jnp.float32),
    mesh=_mesh,
    compiler_params=pltpu.CompilerParams(use_tc_tiling_on_sc=False),
    scratch_types=[
        pltpu.VMEM((CPT, CHUNK), jnp.int32),
        pltpu.VMEM((CHUNK, DEGW), jnp.float32),
        pltpu.VMEM((ROWS_PT, DEGW), jnp.float32),
        pltpu.VMEM_SHARED((N_NODES, DEGW), jnp.float32),
    ],
)


RD = ROWS_PT // NBUF


def _agg_body(g_hbm, src_hbm, dst_hbm, out_hbm,
              src_v, dst_v, rows_v, acc_sh, gsem, ssem):
    cid = lax.axis_index("c")
    sid = lax.axis_index("s")
    wid = cid * NS + sid
    pltpu.sync_copy(src_hbm.at[wid], src_v)
    pltpu.sync_copy(dst_hbm.at[wid], dst_v)

    def fill_zero(r, _):
        for jj in range(HID // 16):
            rows_v[0, r, pl.ds(jj * 16, 16)] = jnp.zeros((16,), jnp.float32)
        return 0

    lax.fori_loop(0, CHUNK, fill_zero, 0)
    for k in range(NBUF):
        pltpu.async_copy(rows_v.at[0, pl.ds(0, RD)],
                         acc_sh.at[pl.ds(sid * ROWS_PT + k * RD, RD)],
                         gsem.at[k])
    for k in range(NBUF):
        pltpu.make_async_copy(rows_v.at[0, pl.ds(0, RD)],
                              acc_sh.at[pl.ds(sid * ROWS_PT + k * RD, RD)],
                              gsem.at[k]).wait()

    @pl.when(sid == 0)
    def _():
        pltpu.sync_copy(rows_v.at[0, pl.ds(0, ROWS_TAIL)],
                        acc_sh.at[pl.ds(NS * ROWS_PT, ROWS_TAIL)])

    plsc.subcore_barrier()

    def start_gather(j, p):
        pltpu.async_copy(g_hbm.at[src_v.at[j]], rows_v.at[p], gsem.at[p])

    def wait_dma(p, sem):
        pltpu.make_async_copy(g_hbm.at[src_v.at[0]], rows_v.at[p],
                              sem.at[p]).wait()

    def start_scatter(j, p):
        pltpu.async_copy(rows_v.at[p], acc_sh.at[dst_v.at[j]], ssem.at[p],
                         add=True)

    for p in range(NBUF):
        start_gather(p, p)

    def step(i, _):
        for p in range(NBUF):
            j = NBUF * i + p
            wait_dma(p, gsem)
            start_scatter(j, p)
        for p in range(NBUF):
            jn = NBUF * i + p + NBUF

            @pl.when(jn <= CPT - 1)
            def _():
                wait_dma(p, ssem)
                start_gather(jn, p)

        return 0

    lax.fori_loop(0, NITER, step, 0)
    for p in range(TAILN, NBUF):
        wait_dma(p, ssem)
    for p in range(TAILN):
        wait_dma(p, gsem)
        start_scatter(NBUF * NITER + p, p)
    for p in range(TAILN):
        wait_dma(p, ssem)
    plsc.subcore_barrier()
    for k in range(NBUF):
        pltpu.async_copy(acc_sh.at[pl.ds(sid * ROWS_PT + k * RD, RD)],
                         rows_v.at[k, pl.ds(0, RD)], gsem.at[k])
    for k in range(NBUF):
        pltpu.make_async_copy(acc_sh.at[pl.ds(sid * ROWS_PT + k * RD, RD)],
                              rows_v.at[k, pl.ds(0, RD)], gsem.at[k]).wait()
        pltpu.async_copy(rows_v.at[k, pl.ds(0, RD)],
                         out_hbm.at[cid, pl.ds(sid * ROWS_PT + k * RD, RD)],
                         ssem.at[k])
    for k in range(NBUF):
        pltpu.make_async_copy(rows_v.at[k, pl.ds(0, RD)],
                              out_hbm.at[cid, pl.ds(sid * ROWS_PT + k * RD, RD)],
                              ssem.at[k]).wait()

    @pl.when(sid == 0)
    def _():
        pltpu.sync_copy(acc_sh.at[pl.ds(NS * ROWS_PT, ROWS_TAIL)],
                        rows_v.at[0, pl.ds(0, ROWS_TAIL)])
        pltpu.sync_copy(rows_v.at[0, pl.ds(0, ROWS_TAIL)],
                        out_hbm.at[cid, pl.ds(NS * ROWS_PT, ROWS_TAIL)])


_agg_call = pl.kernel(
    _agg_body,
    out_type=jax.ShapeDtypeStruct((NC, N_NODES, HID), jnp.float32),
    mesh=_mesh,
    compiler_params=pltpu.CompilerParams(use_tc_tiling_on_sc=False),
    scratch_types=[
        pltpu.VMEM((CPT, CHUNK), jnp.int32),
        pltpu.VMEM((CPT, CHUNK), jnp.int32),
        pltpu.VMEM((NBUF, CHUNK, HID), jnp.float32),
        pltpu.VMEM_SHARED((N_NODES, HID), jnp.float32),
        pltpu.SemaphoreType.DMA((NBUF,)),
        pltpu.SemaphoreType.DMA((NBUF,)),
    ],
)


RBLK = 2000


def _first_body(x_ref, w_ref, deg_ref, g_ref, dinv_ref):
    deg = deg_ref[0, :, 0:1] + deg_ref[1, :, 0:1] + 1.0
    dinv = lax.rsqrt(deg)
    g_ref[...] = jnp.dot(x_ref[...], w_ref[...],
                         preferred_element_type=jnp.float32) * dinv
    dinv_ref[...] = dinv


def _tc_first(x, W1, degp):
    return pl.pallas_call(
        _first_body,
        grid=(N_NODES // RBLK,),
        in_specs=[
            pl.BlockSpec((RBLK, D_IN), lambda i: (i, 0)),
            pl.BlockSpec((D_IN, HID), lambda i: (0, 0)),
            pl.BlockSpec((NC, RBLK, DEGW), lambda i: (0, i, 0)),
        ],
        out_specs=[
            pl.BlockSpec((RBLK, HID), lambda i: (i, 0)),
            pl.BlockSpec((RBLK, 1), lambda i: (i, 0)),
        ],
        out_shape=[
            jax.ShapeDtypeStruct((N_NODES, HID), jnp.float32),
            jax.ShapeDtypeStruct((N_NODES, 1), jnp.float32),
        ],
    )(x, W1, degp)


def _mid_body(acc_ref, g_ref, dinv_ref, b_ref, w_ref, out_ref):
    s = acc_ref[0] + acc_ref[1] + g_ref[...]
    z = jnp.maximum(s * dinv_ref[...] + b_ref[...], 0.0)
    out_ref[...] = jnp.dot(z, w_ref[...],
                           preferred_element_type=jnp.float32) * dinv_ref[...]


def _tc_mid(acc, g_prev, dinv, b_row, W_next):
    return pl.pallas_call(
        _mid_body,
        grid=(N_NODES // RBLK,),
        in_specs=[
            pl.BlockSpec((NC, RBLK, HID), lambda i: (0, i, 0)),
            pl.BlockSpec((RBLK, HID), lambda i: (i, 0)),
            pl.BlockSpec((RBLK, 1), lambda i: (i, 0)),
            pl.BlockSpec((1, HID), lambda i: (0, 0)),
            pl.BlockSpec((HID, HID), lambda i: (0, 0)),
        ],
        out_specs=pl.BlockSpec((RBLK, HID), lambda i: (i, 0)),
        out_shape=jax.ShapeDtypeStruct((N_NODES, HID), jnp.float32),
    )(acc, g_prev, dinv, b_row, W_next)


ROWS_W = 320
NGROUPS = ROWS_W // 16
TAIL_ROWS = N_NODES - (NW - 1) * ROWS_W
TAIL_OFF = ROWS_W - TAIL_ROWS


def _pool_body(acc_hbm, g_hbm, dinv_hbm, b3_hbm, batch_hbm,
               sums_out, maxs_out, cnts_out,
               a0_v, a1_v, g_v, dinv_v, batch_v, b3_v, sum_t, max_t, cnt_t):
    cid = lax.axis_index("c")
    sid = lax.axis_index("s")
    w = cid * NS + sid
    is_last = w == NW - 1
    base = jnp.where(is_last, N_NODES - ROWS_W, w * ROWS_W)
    local_off = jnp.where(is_last, TAIL_OFF, 0)
    ngroups = jnp.where(is_last, TAIL_ROWS // 16, NGROUPS)
    pltpu.sync_copy(acc_hbm.at[0, pl.ds(base, ROWS_W)], a0_v)
    pltpu.sync_copy(acc_hbm.at[1, pl.ds(base, ROWS_W)], a1_v)
    pltpu.sync_copy(g_hbm.at[pl.ds(base, ROWS_W)], g_v)
    pltpu.sync_copy(dinv_hbm.at[pl.ds(base, ROWS_W)], dinv_v)
    pltpu.sync_copy(batch_hbm.at[pl.ds(base, ROWS_W)], batch_v)
    pltpu.sync_copy(b3_hbm, b3_v)

    def zt(r, _):
        for j in range(HID // 16):
            sum_t[r, pl.ds(16 * j, 16)] = jnp.zeros((16,), jnp.float32)
            max_t[r, pl.ds(16 * j, 16)] = jnp.zeros((16,), jnp.float32)
        cnt_t[r, pl.ds(0, 16)] = jnp.zeros((16,), jnp.float32)
        return 0

    lax.fori_loop(0, N_GRAPHS, zt, 0)
    ones16 = jnp.ones((16,), jnp.float32)

    def group(k, _):
        off = local_off + 16 * k
        bvec = batch_v[pl.ds(off, 16)]
        dvec = dinv_v[pl.ds(off, 16)]
        for t in range(16):
            b = bvec[t]
            dv = jnp.broadcast_to(dvec[t], (16,))
            r = off + t
            plsc.addupdate(cnt_t.at[b], ones16)
            for j in range(HID // 16):
                sl = pl.ds(16 * j, 16)
                s = a0_v[r, sl] + a1_v[r, sl] + g_v[r, sl]
                z = jnp.maximum(s * dv + b3_v[sl], 0.0)
                plsc.addupdate(sum_t.at[b, sl], z)
                max_t[b, sl] = jnp.maximum(max_t[b, sl], z)
        return 0

    lax.fori_loop(0, ngroups, group, 0)
    pltpu.sync_copy(sum_t, sums_out.at[w])
    pltpu.sync_copy(max_t, maxs_out.at[w])
    pltpu.sync_copy(cnt_t, cnts_out.at[w])


_pool_call = pl.kernel(
    _pool_body,
    out_type=[
        jax.ShapeDtypeStruct((NW, N_GRAPHS, HID), jnp.float32),
        jax.ShapeDtypeStruct((NW, N_GRAPHS, HID), jnp.float32),
        jax.ShapeDtypeStruct((NW, N_GRAPHS, 16), jnp.float32),
    ],
    mesh=_mesh,
    compiler_params=pltpu.CompilerParams(use_tc_tiling_on_sc=False),
    scratch_types=[
        pltpu.VMEM((ROWS_W, HID), jnp.float32),
        pltpu.VMEM((ROWS_W, HID), jnp.float32),
        pltpu.VMEM((ROWS_W, HID), jnp.float32),
        pltpu.VMEM((ROWS_W,), jnp.float32),
        pltpu.VMEM((ROWS_W,), jnp.int32),
        pltpu.VMEM((HID,), jnp.float32),
        pltpu.VMEM((N_GRAPHS, HID), jnp.float32),
        pltpu.VMEM((N_GRAPHS, HID), jnp.float32),
        pltpu.VMEM((N_GRAPHS, 16), jnp.float32),
    ],
)


def _head2_body(sums_ref, maxs_ref, cnts_ref, meta_ref, wm_ref, bm_ref,
                wp1_ref, bp1_ref, wp2_ref, bp2_ref, out_ref):
    sums = jnp.sum(sums_ref[...], axis=0)
    mx = jnp.max(maxs_ref[...], axis=0)
    counts = jnp.sum(cnts_ref[...], axis=0)[:, 0:1]
    mean = sums / jnp.maximum(counts, 1.0)
    meta_e = jnp.maximum(
        jnp.dot(meta_ref[...], wm_ref[...],
                preferred_element_type=jnp.float32) + bm_ref[...], 0.0)
    fused = jnp.concatenate([mean, mx, meta_e], axis=1)
    hp = jnp.maximum(
        jnp.dot(fused, wp1_ref[...],
                preferred_element_type=jnp.float32) + bp1_ref[...], 0.0)
    out_ref[...] = jnp.dot(hp, wp2_ref[...],
                           preferred_element_type=jnp.float32) + bp2_ref[...]


def _tc_head2(sums_p, maxs_p, cnts_p, metadata, Wm, bm_row,
              Wp1, bp1_row, Wp2, bp2_row):
    return pl.pallas_call(
        _head2_body,
        out_shape=jax.ShapeDtypeStruct((N_GRAPHS, 1), jnp.float32),
    )(sums_p, maxs_p, cnts_p, metadata, Wm, bm_row,
      Wp1, bp1_row, Wp2, bp2_row)


NGRID = N_NODES // RBLK


def _head_body(acc_ref, g_ref, dinv_ref, b_ref, batch_ref, meta_ref,
               wm_ref, bm_ref, wp1_ref, bp1_ref, wp2_ref, bp2_ref, out_ref,
               sums_s, counts_s, maxes_s):
    i = pl.program_id(0)
    s = acc_ref[0] + acc_ref[1] + g_ref[...]
    z = jnp.maximum(s * dinv_ref[...] + b_ref[...], 0.0)
    gids = lax.broadcasted_iota(jnp.int32, (1, N_GRAPHS), 1)
    onehot = (batch_ref[...] == gids).astype(jnp.float32)
    bsums = lax.dot_general(onehot, z, (((0,), (0,)), ((), ())),
                            preferred_element_type=jnp.float32)
    bcounts = lax.dot_general(onehot, jnp.ones((RBLK, 1), jnp.float32),
                              (((0,), (0,)), ((), ())),
                              preferred_element_type=jnp.float32)
    rows = []
    neg = jnp.float32(-3.0e38)
    for g in range(N_GRAPHS):
        mask = batch_ref[...] == g
        m = jnp.max(jnp.where(mask, z, neg), axis=0, keepdims=True)
        rows.append(m)
    bmax = jnp.concatenate(rows, axis=0)

    @pl.when(i == 0)
    def _():
        sums_s[...] = bsums
        counts_s[...] = bcounts
        maxes_s[...] = bmax

    @pl.when(i > 0)
    def _():
        sums_s[...] += bsums
        counts_s[...] += bcounts
        maxes_s[...] = jnp.maximum(maxes_s[...], bmax)

    @pl.when(i == NGRID - 1)
    def _():
        counts = counts_s[...]
        mean = sums_s[...] / jnp.maximum(counts, 1.0)
        mx = jnp.where(counts > 0, maxes_s[...], 0.0)
        meta_e = jnp.maximum(
            jnp.dot(meta_ref[...], wm_ref[...],
                    preferred_element_type=jnp.float32) + bm_ref[...], 0.0)
        fused = jnp.concatenate([mean, mx, meta_e], axis=1)
        hp = jnp.maximum(
            jnp.dot(fused, wp1_ref[...],
                    preferred_element_type=jnp.float32) + bp1_ref[...], 0.0)
        out_ref[...] = jnp.dot(hp, wp2_ref[...],
                               preferred_element_type=jnp.float32) + bp2_ref[...]


def _tc_head(acc, g3, dinv, b_row, batch2d, metadata, Wm, bm_row,
             Wp1, bp1_row, Wp2, bp2_row):
    return pl.pallas_call(
        _head_body,
        grid=(NGRID,),
        in_specs=[
            pl.BlockSpec((NC, RBLK, HID), lambda i: (0, i, 0)),
            pl.BlockSpec((RBLK, HID), lambda i: (i, 0)),
            pl.BlockSpec((RBLK, 1), lambda i: (i, 0)),
            pl.BlockSpec((1, HID), lambda i: (0, 0)),
            pl.BlockSpec((RBLK, 1), lambda i: (i, 0)),
            pl.BlockSpec((N_GRAPHS, 32), lambda i: (0, 0)),
            pl.BlockSpec((32, HID), lambda i: (0, 0)),
            pl.BlockSpec((1, HID), lambda i: (0, 0)),
            pl.BlockSpec((2 * HID + 64, 64), lambda i: (0, 0)),
            pl.BlockSpec((1, 64), lambda i: (0, 0)),
            pl.BlockSpec((64, 1), lambda i: (0, 0)),
            pl.BlockSpec((1, 1), lambda i: (0, 0)),
        ],
        out_specs=pl.BlockSpec((N_GRAPHS, 1), lambda i: (0, 0)),
        out_shape=jax.ShapeDtypeStruct((N_GRAPHS, 1), jnp.float32),
        scratch_shapes=[
            pltpu.VMEM((N_GRAPHS, HID), jnp.float32),
            pltpu.VMEM((N_GRAPHS, 1), jnp.float32),
            pltpu.VMEM((N_GRAPHS, HID), jnp.float32),
        ],
    )(acc, g3, dinv, b_row, batch2d, metadata, Wm, bm_row,
      Wp1, bp1_row, Wp2, bp2_row)


def kernel(x, edge_index, batch, metadata, W1, b1, W2, b2, W3, b3,
           Wm, bm, Wp1, bp1, Wp2, bp2):
    src_r = edge_index[0].reshape(NW, CPT, CHUNK)
    dst_r = edge_index[1].reshape(NW, CPT, CHUNK)
    batch2d = batch.reshape(N_NODES, 1)
    b1r = b1.reshape(1, HID)
    b2r = b2.reshape(1, HID)
    b3r = b3.reshape(1, HID)
    bmr = bm.reshape(1, HID)
    bp1r = bp1.reshape(1, 64)
    bp2r = bp2.reshape(1, 1)

    degp = _deg_call(dst_r)
    g1, dinv = _tc_first(x, W1, degp)
    acc1 = _agg_call(g1, src_r, dst_r)
    g2 = _tc_mid(acc1, g1, dinv, b1r, W2)
    acc2 = _agg_call(g2, src_r, dst_r)
    g3 = _tc_mid(acc2, g2, dinv, b2r, W3)
    acc3 = _agg_call(g3, src_r, dst_r)
    sums_p, maxs_p, cnts_p = _pool_call(acc3, g3, dinv.reshape(N_NODES),
                                        b3, batch)
    return _tc_head2(sums_p, maxs_p, cnts_p, metadata, Wm, bmr,
                     Wp1, bp1r, Wp2, bp2r)
